# baseline (speedup 1.0000x reference)
import functools

import jax
import jax.numpy as jnp
from jax import lax
from jax.experimental import pallas as pl
from jax.experimental.pallas import tpu as pltpu
from jax.experimental.pallas import tpu_sc as plsc

VOCAB = 1000000
NROW = 16384
NCOL = 26
D = 32
NC, NS = 2, 16
NW = NC * NS
RPW = NROW // NW
NT = RPW // 128
SBLK = NROW * 8
TB = D * RPW

_mesh = plsc.VectorSubcoreMesh(core_axis_name="c", subcore_axis_name="s")


@functools.partial(
    pl.kernel,
    out_type=jax.ShapeDtypeStruct((NCOL, D * NROW), jnp.float32),
    mesh=_mesh,
    scratch_types=[
        pltpu.VMEM((RPW, NCOL), jnp.int32),
        pltpu.VMEM((RPW,), jnp.int32),
        pltpu.VMEM((RPW,), jnp.int32),
        pltpu.VMEM((RPW, D), jnp.float32),
        pltpu.VMEM((RPW, D), jnp.float32),
        pltpu.VMEM((TB,), jnp.float32),
        pltpu.VMEM((TB,), jnp.float32),
        pltpu.SemaphoreType.DMA,
        pltpu.SemaphoreType.DMA,
        pltpu.SemaphoreType.DMA,
        pltpu.SemaphoreType.DMA,
    ],
    compiler_params=pltpu.CompilerParams(use_tc_tiling_on_sc=False,
                                         needs_layout_passes=False),
)
def _embed_sc(idx_hbm, w_hbm, out_hbm, idx_v, col0, col1, gbuf0, gbuf1,
              tbuf0, tbuf1, g0, g1, o0, o1):
    wid = lax.axis_index("s") * NC + lax.axis_index("c")
    ib = wid * RPW
    ob = wid * (RPW * 8)
    iota = lax.iota(jnp.int32, 16)

    pltpu.sync_copy(idx_hbm.at[pl.ds(ib, RPW)], idx_v)

    cols = (col0, col1)
    gbufs = (gbuf0, gbuf1)
    tbufs = (tbuf0, tbuf1)
    gsems = (g0, g1)
    osems = (o0, o1)

    def extract(j, col_ref):
        cix = jnp.zeros((16,), jnp.int32) + j

        def body(k, c):
            for u in range(4):
                rows = (k * 4 + u) * 16 + iota
                col_ref[pl.ds((k * 4 + u) * 16, 16)] = plsc.load_gather(
                    idx_v, [rows, cix])
            return c

        lax.fori_loop(0, RPW // 64, body, 0)

    def transpose(gbuf, tbuf):
        def body(k0, c):
            for u in range(4):
                k = k0 * 4 + u
                s = k >> 8
                v = (k >> 6) & (NT - 1)
                dl = (k >> 3) & 7
                w0 = (k & 7) * 16
                rows = v * 128 + w0 + iota
                cix = jnp.zeros((16,), jnp.int32) + (s * 8 + dl)
                vals = plsc.load_gather(gbuf, [rows, cix])
                tbuf[pl.ds(s * 4096 + v * 1024 + dl * 128 + w0, 16)] = vals
            return c

        lax.fori_loop(0, TB // 64, body, 0)

    def writeout(j, tbuf, sem):
        return [
            pltpu.async_copy(
                tbuf.at[pl.ds(s * 4096, 4096)],
                out_hbm.at[j, pl.ds(s * SBLK + ob, 4096)], sem)
            for s in range(D // 8)
        ]

    gather = [None] * NCOL
    outcp = [None] * NCOL
    extract(0, cols[0])
    gather[0] = pltpu.async_copy(w_hbm.at[cols[0]], gbufs[0], gsems[0])
    for j in range(NCOL):
        p = j % 2
        q = (j + 1) % 2
        if j + 1 < NCOL:
            extract(j + 1, cols[q])
            if j >= 1:
                for h in outcp[j - 1]:
                    h.wait()
            gather[j + 1] = pltpu.async_copy(
                w_hbm.at[cols[q]], gbufs[q], gsems[q])
        gather[j].wait()
        transpose(gbufs[p], tbufs[p])
        outcp[j] = writeout(j, tbufs[p], osems[p])
    for h in outcp[NCOL - 2]:
        h.wait()
    for h in outcp[NCOL - 1]:
        h.wait()


def kernel(idx, weight):
    out4 = _embed_sc(idx.astype(jnp.int32), weight)
    out5 = out4.reshape(NCOL, D // 8, NROW // 128, 8, 128)
    out6 = out5.transpose(2, 4, 0, 1, 3)
    return out6.reshape(NROW, NCOL, D)


# speedup vs baseline: 1.4294x; 1.4294x over previous
import functools

import jax
import jax.numpy as jnp
from jax import lax
from jax.experimental import pallas as pl
from jax.experimental.pallas import tpu as pltpu
from jax.experimental.pallas import tpu_sc as plsc

VOCAB = 1000000
NROW = 16384
NCOL = 26
D = 32
NC, NS = 2, 16
NW = NC * NS
RPW = NROW // NW
NT = RPW // 128
WPAD = 133

_mesh = plsc.VectorSubcoreMesh(core_axis_name="c", subcore_axis_name="s")


@functools.partial(
    pl.kernel,
    out_type=jax.ShapeDtypeStruct((NCOL, D // 8, NROW // 128, 8, 128),
                                  jnp.float32),
    mesh=_mesh,
    scratch_types=[
        pltpu.VMEM((RPW, NCOL), jnp.int32),
        pltpu.VMEM((RPW,), jnp.int32),
        pltpu.VMEM((RPW,), jnp.int32),
        pltpu.VMEM((RPW, D), jnp.float32),
        pltpu.VMEM((RPW, D), jnp.float32),
        pltpu.VMEM((D // 8, NT, 8, WPAD), jnp.float32),
        pltpu.VMEM((D // 8, NT, 8, WPAD), jnp.float32),
        pltpu.SemaphoreType.DMA,
        pltpu.SemaphoreType.DMA,
        pltpu.SemaphoreType.DMA,
        pltpu.SemaphoreType.DMA,
    ],
    compiler_params=pltpu.CompilerParams(use_tc_tiling_on_sc=False,
                                         needs_layout_passes=False),
)
def _embed_sc(idx_hbm, w_hbm, out_hbm, idx_v, col0, col1, gbuf0, gbuf1,
              tbuf0, tbuf1, g0, g1, o0, o1):
    wid = lax.axis_index("s") * NC + lax.axis_index("c")
    ib = wid * RPW
    t0 = wid * NT
    iota = lax.iota(jnp.int32, 16)
    zero16 = jnp.zeros((16,), jnp.int32)
    s_lo = lax.shift_right_logical(iota, 3)
    s_hi = s_lo + 2
    dl_ix = lax.bitwise_and(iota, 7)

    pltpu.sync_copy(idx_hbm.at[pl.ds(ib, RPW)], idx_v)

    cols = (col0, col1)
    gbufs = (gbuf0, gbuf1)
    tbufs = (tbuf0, tbuf1)
    gsems = (g0, g1)
    osems = (o0, o1)

    def extract(j, col_ref):
        cix = zero16 + j

        def body(k, c):
            for u in range(4):
                rows = (k * 4 + u) * 16 + iota
                col_ref[pl.ds((k * 4 + u) * 16, 16)] = plsc.load_gather(
                    idx_v, [rows, cix])
            return c

        lax.fori_loop(0, RPW // 64, body, 0)

    def transpose(gbuf, tbuf):
        def body(k, c):
            for u in range(4):
                r = k * 4 + u
                v = zero16 + lax.shift_right_logical(r, 7)
                w = zero16 + lax.bitwise_and(r, 127)
                va = gbuf[r, pl.ds(0, 16)]
                vb = gbuf[r, pl.ds(16, 16)]
                plsc.store_scatter(tbuf, [s_lo, v, dl_ix, w], va)
                plsc.store_scatter(tbuf, [s_hi, v, dl_ix, w], vb)
            return c

        lax.fori_loop(0, RPW // 4, body, 0)

    gather = [None] * NCOL
    outcp = [None] * NCOL
    extract(0, cols[0])
    gather[0] = pltpu.async_copy(w_hbm.at[cols[0]], gbufs[0], gsems[0])
    for j in range(NCOL):
        p = j % 2
        q = (j + 1) % 2
        if j + 1 < NCOL:
            extract(j + 1, cols[q])
            if j >= 1:
                outcp[j - 1].wait()
            gather[j + 1] = pltpu.async_copy(
                w_hbm.at[cols[q]], gbufs[q], gsems[q])
        gather[j].wait()
        transpose(gbufs[p], tbufs[p])
        outcp[j] = pltpu.async_copy(
            tbufs[p].at[:, :, :, pl.ds(0, 128)],
            out_hbm.at[j, :, pl.ds(t0, NT)], osems[p])
    outcp[NCOL - 2].wait()
    outcp[NCOL - 1].wait()


def kernel(idx, weight):
    out5 = _embed_sc(idx.astype(jnp.int32), weight)
    out6 = out5.transpose(2, 4, 0, 1, 3)
    return out6.reshape(NROW, NCOL, D)


# speedup vs baseline: 1.4636x; 1.0239x over previous
import functools

import jax
import jax.numpy as jnp
from jax import lax
from jax.experimental import pallas as pl
from jax.experimental.pallas import tpu as pltpu
from jax.experimental.pallas import tpu_sc as plsc

VOCAB = 1000000
NROW = 16384
NCOL = 26
D = 32
NC, NS = 2, 16
NW = NC * NS
RPW = NROW // NW
NT = RPW // 128
WPAD = 133

_mesh = plsc.VectorSubcoreMesh(core_axis_name="c", subcore_axis_name="s")


@functools.partial(
    pl.kernel,
    out_type=jax.ShapeDtypeStruct((NCOL, D // 8, NROW // 128, 8, 128),
                                  jnp.float32),
    mesh=_mesh,
    scratch_types=[
        pltpu.VMEM((NCOL, RPW), jnp.int32),
        pltpu.VMEM((RPW, D), jnp.float32),
        pltpu.VMEM((RPW, D), jnp.float32),
        pltpu.VMEM((D // 8, NT, 8, WPAD), jnp.float32),
        pltpu.VMEM((D // 8, NT, 8, WPAD), jnp.float32),
        pltpu.SemaphoreType.DMA,
        pltpu.SemaphoreType.DMA,
        pltpu.SemaphoreType.DMA,
        pltpu.SemaphoreType.DMA,
    ],
    compiler_params=pltpu.CompilerParams(use_tc_tiling_on_sc=False,
                                         needs_layout_passes=False),
)
def _embed_sc(idx_hbm, w_hbm, out_hbm, idx_v, gbuf0, gbuf1,
              tbuf0, tbuf1, g0, g1, o0, o1):
    wid = lax.axis_index("s") * NC + lax.axis_index("c")
    ib = wid * RPW
    t0 = wid * NT
    iota = lax.iota(jnp.int32, 16)
    zero16 = jnp.zeros((16,), jnp.int32)
    s_lo = lax.shift_right_logical(iota, 3)
    s_hi = s_lo + 2
    dl_ix = lax.bitwise_and(iota, 7)

    pltpu.sync_copy(idx_hbm.at[:, pl.ds(ib, RPW)], idx_v)

    gbufs = (gbuf0, gbuf1)
    tbufs = (tbuf0, tbuf1)
    gsems = (g0, g1)
    osems = (o0, o1)

    def transpose(gbuf, tbuf):
        def body(k, c):
            for u in range(4):
                r = k * 4 + u
                v = zero16 + lax.shift_right_logical(r, 7)
                w = zero16 + lax.bitwise_and(r, 127)
                va = gbuf[r, pl.ds(0, 16)]
                vb = gbuf[r, pl.ds(16, 16)]
                plsc.store_scatter(tbuf, [s_lo, v, dl_ix, w], va)
                plsc.store_scatter(tbuf, [s_hi, v, dl_ix, w], vb)
            return c

        lax.fori_loop(0, RPW // 4, body, 0)

    gather = [None] * NCOL
    outcp = [None] * NCOL
    gather[0] = pltpu.async_copy(w_hbm.at[idx_v.at[0]], gbufs[0], gsems[0])
    for j in range(NCOL):
        p = j % 2
        q = (j + 1) % 2
        if j + 1 < NCOL:
            if j >= 1:
                outcp[j - 1].wait()
            gather[j + 1] = pltpu.async_copy(
                w_hbm.at[idx_v.at[j + 1]], gbufs[q], gsems[q])
        gather[j].wait()
        transpose(gbufs[p], tbufs[p])
        outcp[j] = pltpu.async_copy(
            tbufs[p].at[:, :, :, pl.ds(0, 128)],
            out_hbm.at[j, :, pl.ds(t0, NT)], osems[p])
    outcp[NCOL - 2].wait()
    outcp[NCOL - 1].wait()


def kernel(idx, weight):
    out5 = _embed_sc(idx.astype(jnp.int32).T, weight)
    out6 = out5.transpose(2, 4, 0, 1, 3)
    return out6.reshape(NROW, NCOL, D)
